# baseline (speedup 1.0000x reference)
import jax
import jax.numpy as jnp
from jax import lax
from jax.experimental import pallas as pl
from jax.experimental.pallas import tpu as pltpu
from jax.experimental.pallas import tpu_sc as plsc

_NEG = -1e9

_N = 10000
_E = 160000
_D = 256
_H = 512
_M = 64

_NC = 2
_NS = 16
_NW = _NC * _NS


def _pack_words(v):
    hp = v.shape[1] // 2
    lo = lax.bitcast_convert_type(v[:, :hp].astype(jnp.bfloat16), jnp.uint16)
    hi = lax.bitcast_convert_type(v[:, hp:].astype(jnp.bfloat16), jnp.uint16)
    return lo.astype(jnp.int32) | (hi.astype(jnp.int32) << 16)


def _mm_body(nf_ref, w1a_ref, w1b_ref, b1_ref, ab_ref):
    i = pl.program_id(0)
    half = pl.num_programs(0) // 2
    x = nf_ref[...]
    w = jnp.where(i < half, w1a_ref[...], w1b_ref[...])
    bias = jnp.where(i < half, b1_ref[...], 0.0)
    ab_ref[...] = _pack_words(
        jnp.dot(x, w, preferred_element_type=jnp.float32) + bias
    )


def _node_tables(nf, w1a, w1b, b1):
    bn = 400
    nb = _N // bn
    return pl.pallas_call(
        _mm_body,
        grid=(2 * nb,),
        in_specs=[
            pl.BlockSpec((bn, _D), lambda i: (i % (_N // 400), 0)),
            pl.BlockSpec((_D, _H), lambda i: (0, 0)),
            pl.BlockSpec((_D, _H), lambda i: (0, 0)),
            pl.BlockSpec((1, _H), lambda i: (0, 0)),
        ],
        out_specs=pl.BlockSpec((bn, _H // 2), lambda i: (i, 0)),
        out_shape=jax.ShapeDtypeStruct((2 * _N, _H // 2), jnp.int32),
    )(nf, w1a, w1b, b1)


_CH2 = 40


def _gather_route_body(ab_hbm, cidx_hbm, z2_hbm,
                       civ, cb0, cb1, cb2, cb3, cb4, cb5,
                       sg0, sg1, sg2, sg3, sg4, sg5,
                       ss0, ss1, ss2, ss3, ss4, ss5):
    rows = z2_hbm.shape[0]
    rpw = rows // _NW
    nchunk = rpw // (2 * _CH2)
    wid = lax.axis_index("s") * _NC + lax.axis_index("c")
    base0 = wid * rpw
    nset = 6
    bufs = ((cb0, sg0, ss0), (cb1, sg1, ss1), (cb2, sg2, ss2),
            (cb3, sg3, ss3), (cb4, sg4, ss4), (cb5, sg5, ss5))
    cw = 2 * _CH2

    pltpu.sync_copy(cidx_hbm.at[pl.ds(base0, rpw)], civ)

    def fetch(k, fs):
        cb, sg, ss = bufs[fs]
        pltpu.async_copy(ab_hbm.at[civ.at[pl.ds(k * cw, cw)]], cb, sg)

    def wait_store(fs):
        cb, sg, ss = bufs[fs]
        pltpu.make_async_copy(cb, z2_hbm.at[pl.ds(base0, cw)], ss).wait()

    def process(k, cs, fs):
        cb, sg, ss = bufs[cs]
        pltpu.make_async_copy(ab_hbm.at[civ.at[pl.ds(k * cw, cw)]], cb, sg).wait()

        @pl.when(k >= 3)
        def _():
            wait_store(fs)

        pltpu.async_copy(cb, z2_hbm.at[pl.ds(base0 + k * cw, cw)], ss)

        @pl.when(k + 3 < nchunk)
        def _():
            fetch(k + 3, fs)

    fetch(0, 0)
    fetch(1, 1)
    fetch(2, 2)

    nhex = nchunk // nset

    def hexa(g, _):
        k0 = g * nset
        for j in range(nset):
            process(k0 + j, j, (j + 3) % nset)
        return 0

    lax.fori_loop(0, nhex, hexa, 0)
    for k in range(nhex * nset, nchunk):
        process(k, k % nset, (k + 3) % nset)

    for k in range(max(nchunk - 3, 0), nchunk):
        wait_store(k % nset)


def _gather_route(ab_pk, cidx):
    rows = cidx.shape[0]
    mesh = plsc.VectorSubcoreMesh(core_axis_name="c", subcore_axis_name="s")
    hp = _H // 2
    return pl.kernel(
        _gather_route_body,
        out_type=jax.ShapeDtypeStruct((rows, hp), jnp.int32),
        mesh=mesh,
        scratch_types=[
            pltpu.VMEM((rows // _NW,), jnp.int32),
        ] + [pltpu.VMEM((2 * _CH2, hp), jnp.int32) for _ in range(6)] + [
            pltpu.SemaphoreType.DMA for _ in range(12)
        ],
    )(ab_pk, cidx)


def _mish_dot_body(z2_ref, w2e_ref, w2o_ref, b2_ref, val_ref):
    hi = jnp.int32(-65536)
    w2e = w2e_ref[...]
    w2o = w2o_ref[...]

    def mish(z):
        u = 1.0 + jnp.exp(z)
        return z * (1.0 - 2.0 / (u * u + 1.0))

    npiece = z2_ref.shape[0] // (2 * _CH2)
    for p in range(npiece):
        wa = z2_ref[p * 2 * _CH2 : p * 2 * _CH2 + _CH2, :]
        wb = z2_ref[p * 2 * _CH2 + _CH2 : (p + 1) * 2 * _CH2, :]
        ev = lax.bitcast_convert_type(wa << 16, jnp.float32) + (
            lax.bitcast_convert_type(wb << 16, jnp.float32)
        )
        od = lax.bitcast_convert_type(wa & hi, jnp.float32) + (
            lax.bitcast_convert_type(wb & hi, jnp.float32)
        )
        v = jnp.sum(mish(ev) * w2e, axis=1, keepdims=True)
        v = v + jnp.sum(mish(od) * w2o, axis=1, keepdims=True)
        val_ref[pl.ds(p * _CH2, _CH2), :] = v + b2_ref[0, 0]


def _mish_dot(z2, w2e, w2o, b2):
    be = 1600
    eg = z2.shape[0] // 2
    g = eg // be
    hp = _H // 2
    out = pl.pallas_call(
        _mish_dot_body,
        grid=(g,),
        in_specs=[
            pl.BlockSpec((2 * be, hp), lambda i: (i, 0)),
            pl.BlockSpec((1, hp), lambda i: (0, 0)),
            pl.BlockSpec((1, hp), lambda i: (0, 0)),
            pl.BlockSpec((1, 1), lambda i: (0, 0)),
        ],
        out_specs=pl.BlockSpec((be, 1), lambda i: (i, 0)),
        out_shape=jax.ShapeDtypeStruct((eg, 1), jnp.float32),
    )(z2, w2e, w2o, b2)
    return out.reshape(eg)


_RB = 80


def _mailbox_body(starts_ref, counts_ref, mne_ref, val_ref, out_ref):
    i = pl.program_id(0)
    iot = lax.broadcasted_iota(jnp.int32, (1, _M), 1)
    for r in range(_RB):
        n = i * _RB + r
        s = starts_ref[n]
        sa = pl.multiple_of((s // 128) * 128, 128)
        off = s - sa
        c = jnp.minimum(counts_ref[n], mne_ref[0])
        w = val_ref[pl.ds(0, 1), pl.ds(sa, 256)]
        w = pltpu.roll(w, 256 - off, 1)[:, :_M]
        out_ref[pl.ds(r, 1), :] = jnp.where(iot < c, w, _NEG)


def _mailbox(starts, counts, mne, val_row):
    return pl.pallas_call(
        _mailbox_body,
        grid=(_N // _RB,),
        in_specs=[
            pl.BlockSpec(memory_space=pltpu.SMEM),
            pl.BlockSpec(memory_space=pltpu.SMEM),
            pl.BlockSpec(memory_space=pltpu.SMEM),
            pl.BlockSpec((1, _E + 256), lambda i: (0, 0)),
        ],
        out_specs=pl.BlockSpec((_RB, _M), lambda i: (i, 0)),
        out_shape=jax.ShapeDtypeStruct((_N, _M), jnp.float32),
    )(starts, counts, mne, val_row)


def kernel(node_feature, W1, b1, W2, b2, src_idx, dst_idx, maximum_num_enemy,
           attack_edge_type_index):
    nf = node_feature.astype(jnp.float32)
    dst = dst_idx.astype(jnp.int32)
    src = src_idx.astype(jnp.int32)

    w1a = W1[:_D]
    w1b = W1[_D:]
    b1r = b1.reshape(1, _H)
    w2e_row = W2[: _H // 2].reshape(1, _H // 2)
    w2o_row = W2[_H // 2 :].reshape(1, _H // 2)
    b2r = b2.reshape(1, 1)

    ab_pk = _node_tables(nf, w1a, w1b, b1r)
    cidx = jnp.stack(
        [dst.reshape(-1, _CH2), (src + _N).reshape(-1, _CH2)], axis=1
    ).reshape(-1)
    ng = 5
    eg = _E // ng
    zs = []
    for g in range(ng):
        sl = slice(g * 2 * eg, (g + 1) * 2 * eg)
        zs.append(_gather_route(ab_pk, cidx[sl]))
    vals = [_mish_dot(z2_g, w2e_row, w2o_row, b2r) for z2_g in zs]
    val = jnp.concatenate(vals)

    counts = jnp.bincount(dst, length=_N).astype(jnp.int32)
    starts = (jnp.cumsum(counts) - counts).astype(jnp.int32)
    val_row = jnp.concatenate([val, jnp.zeros((256,), jnp.float32)]).reshape(1, _E + 256)
    mne = jnp.asarray(maximum_num_enemy, jnp.int32).reshape(1)

    return _mailbox(starts, counts, mne, val_row)


# speedup vs baseline: 1.0594x; 1.0594x over previous
import jax
import jax.numpy as jnp
from jax import lax
from jax.experimental import pallas as pl
from jax.experimental.pallas import tpu as pltpu
from jax.experimental.pallas import tpu_sc as plsc

_NEG = -1e9

_N = 10000
_E = 160000
_D = 256
_H = 512
_M = 64

_NC = 2
_NS = 16
_NW = _NC * _NS


def _pack_words(v):
    hp = v.shape[1] // 2
    lo = lax.bitcast_convert_type(v[:, :hp].astype(jnp.bfloat16), jnp.uint16)
    hi = lax.bitcast_convert_type(v[:, hp:].astype(jnp.bfloat16), jnp.uint16)
    return lo.astype(jnp.int32) | (hi.astype(jnp.int32) << 16)


def _mm_body(nf_ref, w1a_ref, w1b_ref, b1_ref, a_ref, b_ref):
    x = nf_ref[...]
    a_ref[...] = _pack_words(
        jnp.dot(x, w1a_ref[...], preferred_element_type=jnp.float32) + b1_ref[...]
    )
    b_ref[...] = _pack_words(
        jnp.dot(x, w1b_ref[...], preferred_element_type=jnp.float32)
    )


def _node_tables(nf, w1a, w1b, b1):
    bn = 400
    return pl.pallas_call(
        _mm_body,
        grid=(_N // bn,),
        in_specs=[
            pl.BlockSpec((bn, _D), lambda i: (i, 0)),
            pl.BlockSpec((_D, _H), lambda i: (0, 0)),
            pl.BlockSpec((_D, _H), lambda i: (0, 0)),
            pl.BlockSpec((1, _H), lambda i: (0, 0)),
        ],
        out_specs=[
            pl.BlockSpec((bn, _H // 2), lambda i: (i, 0)),
            pl.BlockSpec((bn, _H // 2), lambda i: (i, 0)),
        ],
        out_shape=[
            jax.ShapeDtypeStruct((_N, _H // 2), jnp.int32),
            jax.ShapeDtypeStruct((_N, _H // 2), jnp.int32),
        ],
    )(nf, w1a, w1b, b1)


_CH2 = 40


def _gather_route_body(a_hbm, b_hbm, dst_hbm, src_hbm, za_hbm, zb_hbm,
                       dva, sva, ar0, br0, ar1, br1, ar2, br2, ar3, br3,
                       ar4, br4, ar5, br5,
                       sg0, sg1, sg2, sg3, sg4, sg5,
                       ss0, ss1, ss2, ss3, ss4, ss5):
    eg = za_hbm.shape[0]
    epw = eg // _NW
    nchunk = epw // _CH2
    wid = lax.axis_index("s") * _NC + lax.axis_index("c")
    base0 = wid * epw
    nset = 6
    bufs = ((ar0, br0, sg0, ss0), (ar1, br1, sg1, ss1), (ar2, br2, sg2, ss2),
            (ar3, br3, sg3, ss3), (ar4, br4, sg4, ss4), (ar5, br5, sg5, ss5))

    pltpu.sync_copy(dst_hbm.at[pl.ds(base0, epw)], dva)
    pltpu.sync_copy(src_hbm.at[pl.ds(base0, epw)], sva)

    def fetch(k, fs):
        ar, br, sg, ss = bufs[fs]
        off = pl.ds(k * _CH2, _CH2)
        pltpu.async_copy(a_hbm.at[dva.at[off]], ar, sg)
        pltpu.async_copy(b_hbm.at[sva.at[off]], br, sg)

    def wait_stores(fs):
        ar, br, sg, ss = bufs[fs]
        pltpu.make_async_copy(ar, za_hbm.at[pl.ds(base0, _CH2)], ss).wait()
        pltpu.make_async_copy(br, zb_hbm.at[pl.ds(base0, _CH2)], ss).wait()

    def process(k, cs, fs):
        ar, br, sg, ss = bufs[cs]
        off = pl.ds(k * _CH2, _CH2)
        pltpu.make_async_copy(a_hbm.at[dva.at[off]], ar, sg).wait()
        pltpu.make_async_copy(b_hbm.at[sva.at[off]], br, sg).wait()

        @pl.when(k >= 3)
        def _():
            wait_stores(fs)

        sl = pl.ds(base0 + k * _CH2, _CH2)
        pltpu.async_copy(ar, za_hbm.at[sl], ss)
        pltpu.async_copy(br, zb_hbm.at[sl], ss)

        @pl.when(k + 3 < nchunk)
        def _():
            fetch(k + 3, fs)

    fetch(0, 0)
    fetch(1, 1)
    fetch(2, 2)

    nhex = nchunk // nset

    def hexa(g, _):
        k0 = g * nset
        for j in range(nset):
            process(k0 + j, j, (j + 3) % nset)
        return 0

    lax.fori_loop(0, nhex, hexa, 0)
    for k in range(nhex * nset, nchunk):
        process(k, k % nset, (k + 3) % nset)

    for k in range(max(nchunk - 3, 0), nchunk):
        wait_stores(k % nset)


def _gather_route(a_pk, b_pk, dst, src):
    eg = dst.shape[0]
    mesh = plsc.VectorSubcoreMesh(core_axis_name="c", subcore_axis_name="s")
    hp = _H // 2
    return pl.kernel(
        _gather_route_body,
        out_type=(
            jax.ShapeDtypeStruct((eg, hp), jnp.int32),
            jax.ShapeDtypeStruct((eg, hp), jnp.int32),
        ),
        mesh=mesh,
        scratch_types=[
            pltpu.VMEM((eg // _NW,), jnp.int32),
            pltpu.VMEM((eg // _NW,), jnp.int32),
        ] + [pltpu.VMEM((_CH2, hp), jnp.int32) for _ in range(12)] + [
            pltpu.SemaphoreType.DMA for _ in range(12)
        ],
    )(a_pk, b_pk, dst, src)


def _mish_dot_body(za_ref, zb_ref, w2e_ref, w2o_ref, b2_ref, val_ref):
    hi = jnp.int32(-65536)
    wa = za_ref[...]
    wb = zb_ref[...]
    ev = lax.bitcast_convert_type(wa << 16, jnp.float32) + lax.bitcast_convert_type(
        wb << 16, jnp.float32
    )
    od = lax.bitcast_convert_type(wa & hi, jnp.float32) + lax.bitcast_convert_type(
        wb & hi, jnp.float32
    )

    def mish(z):
        u = 1.0 + jnp.exp(z)
        return z * (1.0 - 2.0 / (u * u + 1.0))

    v = jnp.sum(mish(ev) * w2e_ref[...], axis=1, keepdims=True)
    v = v + jnp.sum(mish(od) * w2o_ref[...], axis=1, keepdims=True)
    val_ref[...] = v + b2_ref[0, 0]


def _mish_dot(za, zb, w2e, w2o, b2):
    be = 1600
    eg = za.shape[0]
    g = eg // be
    hp = _H // 2
    out = pl.pallas_call(
        _mish_dot_body,
        grid=(g,),
        in_specs=[
            pl.BlockSpec((be, hp), lambda i: (i, 0)),
            pl.BlockSpec((be, hp), lambda i: (i, 0)),
            pl.BlockSpec((1, hp), lambda i: (0, 0)),
            pl.BlockSpec((1, hp), lambda i: (0, 0)),
            pl.BlockSpec((1, 1), lambda i: (0, 0)),
        ],
        out_specs=pl.BlockSpec((be, 1), lambda i: (i, 0)),
        out_shape=jax.ShapeDtypeStruct((eg, 1), jnp.float32),
    )(za, zb, w2e, w2o, b2)
    return out.reshape(eg)


_RB = 80


def _mailbox_body(starts_ref, counts_ref, mne_ref, val_ref, out_ref):
    i = pl.program_id(0)
    iot = lax.broadcasted_iota(jnp.int32, (1, _M), 1)
    for r in range(_RB):
        n = i * _RB + r
        s = starts_ref[n]
        sa = pl.multiple_of((s // 128) * 128, 128)
        off = s - sa
        c = jnp.minimum(counts_ref[n], mne_ref[0])
        w = val_ref[pl.ds(0, 1), pl.ds(sa, 256)]
        w = pltpu.roll(w, 256 - off, 1)[:, :_M]
        out_ref[pl.ds(r, 1), :] = jnp.where(iot < c, w, _NEG)


def _mailbox(starts, counts, mne, val_row):
    return pl.pallas_call(
        _mailbox_body,
        grid=(_N // _RB,),
        in_specs=[
            pl.BlockSpec(memory_space=pltpu.SMEM),
            pl.BlockSpec(memory_space=pltpu.SMEM),
            pl.BlockSpec(memory_space=pltpu.SMEM),
            pl.BlockSpec((1, _E + 256), lambda i: (0, 0)),
        ],
        out_specs=pl.BlockSpec((_RB, _M), lambda i: (i, 0)),
        out_shape=jax.ShapeDtypeStruct((_N, _M), jnp.float32),
    )(starts, counts, mne, val_row)


def kernel(node_feature, W1, b1, W2, b2, src_idx, dst_idx, maximum_num_enemy,
           attack_edge_type_index):
    nf = node_feature.astype(jnp.float32)
    dst = dst_idx.astype(jnp.int32)
    src = src_idx.astype(jnp.int32)

    w1a = W1[:_D]
    w1b = W1[_D:]
    b1r = b1.reshape(1, _H)
    w2e_row = W2[: _H // 2].reshape(1, _H // 2)
    w2o_row = W2[_H // 2 :].reshape(1, _H // 2)
    b2r = b2.reshape(1, 1)

    a_pk, b_pk = _node_tables(nf, w1a, w1b, b1r)
    ng = 5
    eg = _E // ng
    zs = []
    for g in range(ng):
        sl = slice(g * eg, (g + 1) * eg)
        zs.append(_gather_route(a_pk, b_pk, dst[sl], src[sl]))
    vals = [_mish_dot(za_g, zb_g, w2e_row, w2o_row, b2r) for za_g, zb_g in zs]
    val = jnp.concatenate(vals)

    counts = jnp.bincount(dst, length=_N).astype(jnp.int32)
    starts = (jnp.cumsum(counts) - counts).astype(jnp.int32)
    val_row = jnp.concatenate([val, jnp.zeros((256,), jnp.float32)]).reshape(1, _E + 256)
    mne = jnp.asarray(maximum_num_enemy, jnp.int32).reshape(1)

    return _mailbox(starts, counts, mne, val_row)
